# baseline (speedup 1.0000x reference)
import functools
import jax
import jax.numpy as jnp
from jax import lax
from jax.experimental import pallas as pl
from jax.experimental.pallas import tpu as pltpu
from jax.experimental.pallas import tpu_sc as plsc

_N = 1000
_E = 100
_NP = 1024
_EP = 128

_NC = 2
_NS = 16
_NW = _NC * _NS
_ROWS = _NP // _NW


def _sc_build_operators(edge_hbm, ops_hbm, ev, slab):
    wid = lax.axis_index("s") * _NC + lax.axis_index("c")
    base = wid * _ROWS
    pltpu.sync_copy(edge_hbm, ev)

    one = jnp.ones((16,), jnp.float32)
    zero = jnp.zeros((16,), jnp.float32)

    svec = [ev[pl.ds(j * 16, 16)] for j in range(_EP // 16)]
    dvec = [ev[pl.ds(_EP + j * 16, 16)] for j in range(_EP // 16)]
    for r in range(_ROWS):
        node = base + r
        row = r * 2 * _EP
        for j in range(_EP // 16):
            slab[pl.ds(row + j * 16, 16)] = jnp.where(svec[j] == node, one, zero)
            slab[pl.ds(row + _EP + j * 16, 16)] = jnp.where(dvec[j] == node, one, zero)
    pltpu.sync_copy(slab, ops_hbm.at[pl.ds(base * 2 * _EP, _ROWS * 2 * _EP)])


_sc_prep = functools.partial(
    pl.kernel,
    out_type=jax.ShapeDtypeStruct((_NP * 2 * _EP,), jnp.float32),
    mesh=plsc.VectorSubcoreMesh(core_axis_name="c", subcore_axis_name="s"),
    scratch_types=[
        pltpu.VMEM((2 * _EP,), jnp.int32),
        pltpu.VMEM((_ROWS * 2 * _EP,), jnp.float32),
    ],
)(_sc_build_operators)


def _fused_body(ops_ref, x_ref,
                wl0, bl0, wr0, wl1, bl1, wr1, wl2, bl2, wr2,
                wl3, bl3, wr3, wl4, bl4, wr4, out_ref):
    ops = ops_ref[...]
    p_src_t = ops[:, :_EP]
    p_dst = ops[:, _EP:]
    count = jnp.sum(p_dst, axis=1, keepdims=True)
    p_dst = p_dst / jnp.maximum(count, 1.0)

    weights = ((wl0, bl0, wr0), (wl1, bl1, wr1), (wl2, bl2, wr2),
               (wl3, bl3, wr3), (wl4, bl4, wr4))

    h = x_ref[...]
    for i, (wl, bl, wr) in enumerate(weights):
        if i > 0:
            h = jnp.maximum(h, 0.0)
        xj = lax.dot_general(p_src_t, h, (((0,), (0,)), ((), ())),
                             preferred_element_type=jnp.float32)
        m = lax.dot_general(xj, wl[...], (((1,), (1,)), ((), ())),
                            preferred_element_type=jnp.float32)
        aggl = lax.dot_general(p_dst, m, (((1,), (0,)), ((), ())),
                               preferred_element_type=jnp.float32)
        dense = lax.dot_general(h, wr[...], (((1,), (1,)), ((), ())),
                                preferred_element_type=jnp.float32)
        h = aggl + dense + bl[...]
    out_ref[...] = h


def kernel(L_x_, L_edge_index_, L_self_modules_convs_modules_0_modules_lin_l_parameters_weight_, L_self_modules_convs_modules_0_modules_lin_l_parameters_bias_, L_self_modules_convs_modules_0_modules_lin_r_parameters_weight_, L_self_modules_convs_modules_1_modules_lin_l_parameters_weight_, L_self_modules_convs_modules_1_modules_lin_l_parameters_bias_, L_self_modules_convs_modules_1_modules_lin_r_parameters_weight_, L_self_modules_convs_modules_2_modules_lin_l_parameters_weight_, L_self_modules_convs_modules_2_modules_lin_l_parameters_bias_, L_self_modules_convs_modules_2_modules_lin_r_parameters_weight_, L_self_modules_convs_modules_3_modules_lin_l_parameters_weight_, L_self_modules_convs_modules_3_modules_lin_l_parameters_bias_, L_self_modules_convs_modules_3_modules_lin_r_parameters_weight_, L_self_modules_convs_modules_4_modules_lin_l_parameters_weight_, L_self_modules_convs_modules_4_modules_lin_l_parameters_bias_, L_self_modules_convs_modules_4_modules_lin_r_parameters_weight_):
    x = L_x_
    edge = L_edge_index_.astype(jnp.int32)
    edge_p = jnp.pad(edge, ((0, 0), (0, _EP - _E)), constant_values=-1)

    ops_flat = _sc_prep(edge_p.reshape(-1))
    ops = ops_flat.reshape(_NP, 2 * _EP)

    x_p = jnp.zeros((_NP, 128), dtype=jnp.float32).at[:_N, :].set(x)

    ws = [
        L_self_modules_convs_modules_0_modules_lin_l_parameters_weight_,
        L_self_modules_convs_modules_0_modules_lin_l_parameters_bias_.reshape(1, -1),
        L_self_modules_convs_modules_0_modules_lin_r_parameters_weight_,
        L_self_modules_convs_modules_1_modules_lin_l_parameters_weight_,
        L_self_modules_convs_modules_1_modules_lin_l_parameters_bias_.reshape(1, -1),
        L_self_modules_convs_modules_1_modules_lin_r_parameters_weight_,
        L_self_modules_convs_modules_2_modules_lin_l_parameters_weight_,
        L_self_modules_convs_modules_2_modules_lin_l_parameters_bias_.reshape(1, -1),
        L_self_modules_convs_modules_2_modules_lin_r_parameters_weight_,
        L_self_modules_convs_modules_3_modules_lin_l_parameters_weight_,
        L_self_modules_convs_modules_3_modules_lin_l_parameters_bias_.reshape(1, -1),
        L_self_modules_convs_modules_3_modules_lin_r_parameters_weight_,
        L_self_modules_convs_modules_4_modules_lin_l_parameters_weight_,
        L_self_modules_convs_modules_4_modules_lin_l_parameters_bias_.reshape(1, -1),
        L_self_modules_convs_modules_4_modules_lin_r_parameters_weight_,
    ]

    out = pl.pallas_call(
        _fused_body,
        out_shape=jax.ShapeDtypeStruct((_NP, 256), jnp.float32),
    )(ops, x_p, *ws)
    return out[:_N]


# speedup vs baseline: 1.0615x; 1.0615x over previous
import functools
import jax
import jax.numpy as jnp
from jax import lax
from jax.experimental import pallas as pl
from jax.experimental.pallas import tpu as pltpu
from jax.experimental.pallas import tpu_sc as plsc

_N = 1000
_E = 100
_NP = 1024
_EP = 128

_NC = 2
_NS = 16
_NW = _NC * _NS
_ROWS = _NP // _NW


def _sc_build_operators(edge_hbm, ops_hbm, ev, slab):
    wid = lax.axis_index("s") * _NC + lax.axis_index("c")
    base = wid * _ROWS
    pltpu.sync_copy(edge_hbm, ev)

    one = jnp.ones((16,), jnp.float32)
    zero = jnp.zeros((16,), jnp.float32)

    svec = [ev[pl.ds(j * 16, 16)] for j in range(_EP // 16)]
    dvec = [ev[pl.ds(_EP + j * 16, 16)] for j in range(_EP // 16)]
    for r in range(_ROWS):
        node = base + r
        row = r * 2 * _EP
        for j in range(_EP // 16):
            slab[pl.ds(row + j * 16, 16)] = jnp.where(svec[j] == node, one, zero)
            slab[pl.ds(row + _EP + j * 16, 16)] = jnp.where(dvec[j] == node, one, zero)
    pltpu.sync_copy(slab, ops_hbm.at[pl.ds(base * 2 * _EP, _ROWS * 2 * _EP)])


_sc_prep = functools.partial(
    pl.kernel,
    out_type=jax.ShapeDtypeStruct((_NP * 2 * _EP,), jnp.float32),
    mesh=plsc.VectorSubcoreMesh(core_axis_name="c", subcore_axis_name="s"),
    scratch_types=[
        pltpu.VMEM((2 * _EP,), jnp.int32),
        pltpu.VMEM((_ROWS * 2 * _EP,), jnp.float32),
    ],
)(_sc_build_operators)


def _fused_body(ops_ref, x_ref,
                wl0, bl0, wr0, wl1, bl1, wr1, wl2, bl2, wr2,
                wl3, bl3, wr3, wl4, bl4, wr4, out_ref):
    ops = ops_ref[...]
    p_src_t = ops[:, :_EP]
    p_dst = ops[:, _EP:]
    count = jnp.sum(p_dst, axis=1, keepdims=True)
    p_dst = p_dst / jnp.maximum(count, 1.0)

    weights = ((wl0, bl0, wr0), (wl1, bl1, wr1), (wl2, bl2, wr2),
               (wl3, bl3, wr3), (wl4, bl4, wr4))

    h = jnp.concatenate(
        [x_ref[...], jnp.zeros((_NP - _N, 128), jnp.float32)], axis=0)
    for i, (wl, bl, wr) in enumerate(weights):
        if i > 0:
            h = jnp.maximum(h, 0.0)
        xj = lax.dot_general(p_src_t, h, (((0,), (0,)), ((), ())),
                             preferred_element_type=jnp.float32)
        m = lax.dot_general(xj, wl[...], (((1,), (1,)), ((), ())),
                            preferred_element_type=jnp.float32)
        aggl = lax.dot_general(p_dst, m, (((1,), (0,)), ((), ())),
                               preferred_element_type=jnp.float32)
        dense = lax.dot_general(h, wr[...], (((1,), (1,)), ((), ())),
                                preferred_element_type=jnp.float32)
        h = aggl + dense + bl[...]
    out_ref[...] = h[:_N]


def kernel(L_x_, L_edge_index_, L_self_modules_convs_modules_0_modules_lin_l_parameters_weight_, L_self_modules_convs_modules_0_modules_lin_l_parameters_bias_, L_self_modules_convs_modules_0_modules_lin_r_parameters_weight_, L_self_modules_convs_modules_1_modules_lin_l_parameters_weight_, L_self_modules_convs_modules_1_modules_lin_l_parameters_bias_, L_self_modules_convs_modules_1_modules_lin_r_parameters_weight_, L_self_modules_convs_modules_2_modules_lin_l_parameters_weight_, L_self_modules_convs_modules_2_modules_lin_l_parameters_bias_, L_self_modules_convs_modules_2_modules_lin_r_parameters_weight_, L_self_modules_convs_modules_3_modules_lin_l_parameters_weight_, L_self_modules_convs_modules_3_modules_lin_l_parameters_bias_, L_self_modules_convs_modules_3_modules_lin_r_parameters_weight_, L_self_modules_convs_modules_4_modules_lin_l_parameters_weight_, L_self_modules_convs_modules_4_modules_lin_l_parameters_bias_, L_self_modules_convs_modules_4_modules_lin_r_parameters_weight_):
    x = L_x_
    edge = L_edge_index_.astype(jnp.int32)
    edge_p = jnp.pad(edge, ((0, 0), (0, _EP - _E)), constant_values=-1)

    ops_flat = _sc_prep(edge_p.reshape(-1))
    ops = ops_flat.reshape(_NP, 2 * _EP)

    ws = [
        L_self_modules_convs_modules_0_modules_lin_l_parameters_weight_,
        L_self_modules_convs_modules_0_modules_lin_l_parameters_bias_.reshape(1, -1),
        L_self_modules_convs_modules_0_modules_lin_r_parameters_weight_,
        L_self_modules_convs_modules_1_modules_lin_l_parameters_weight_,
        L_self_modules_convs_modules_1_modules_lin_l_parameters_bias_.reshape(1, -1),
        L_self_modules_convs_modules_1_modules_lin_r_parameters_weight_,
        L_self_modules_convs_modules_2_modules_lin_l_parameters_weight_,
        L_self_modules_convs_modules_2_modules_lin_l_parameters_bias_.reshape(1, -1),
        L_self_modules_convs_modules_2_modules_lin_r_parameters_weight_,
        L_self_modules_convs_modules_3_modules_lin_l_parameters_weight_,
        L_self_modules_convs_modules_3_modules_lin_l_parameters_bias_.reshape(1, -1),
        L_self_modules_convs_modules_3_modules_lin_r_parameters_weight_,
        L_self_modules_convs_modules_4_modules_lin_l_parameters_weight_,
        L_self_modules_convs_modules_4_modules_lin_l_parameters_bias_.reshape(1, -1),
        L_self_modules_convs_modules_4_modules_lin_r_parameters_weight_,
    ]

    return pl.pallas_call(
        _fused_body,
        out_shape=jax.ShapeDtypeStruct((_N, 256), jnp.float32),
    )(ops, x, *ws)


# speedup vs baseline: 1.0689x; 1.0069x over previous
import functools
import jax
import jax.numpy as jnp
from jax import lax
from jax.experimental import pallas as pl
from jax.experimental.pallas import tpu as pltpu
from jax.experimental.pallas import tpu_sc as plsc

_N = 1000
_E = 100
_NP = 1024
_EP = 128

_NC = 2
_NS = 16
_NW = _NC * _NS
_ROWS = _NP // _NW


def _sc_build_operators(edge_hbm, ops_hbm, ev, slab):
    wid = lax.axis_index("s") * _NC + lax.axis_index("c")
    base = wid * _ROWS
    pltpu.sync_copy(edge_hbm, ev)

    one = jnp.ones((16,), jnp.float32)
    zero = jnp.zeros((16,), jnp.float32)

    svec = [ev[pl.ds(j * 16, 16)] for j in range(_EP // 16)]
    dvec = [ev[pl.ds(_EP + j * 16, 16)] for j in range(_EP // 16)]
    for r in range(_ROWS):
        node = base + r
        row = r * 2 * _EP
        for j in range(_EP // 16):
            slab[pl.ds(row + j * 16, 16)] = jnp.where(svec[j] == node, one, zero)
            slab[pl.ds(row + _EP + j * 16, 16)] = jnp.where(dvec[j] == node, one, zero)
    pltpu.sync_copy(slab, ops_hbm.at[pl.ds(base * 2 * _EP, _ROWS * 2 * _EP)])


_sc_prep = functools.partial(
    pl.kernel,
    out_type=jax.ShapeDtypeStruct((_NP * 2 * _EP,), jnp.float32),
    mesh=plsc.VectorSubcoreMesh(core_axis_name="c", subcore_axis_name="s"),
    scratch_types=[
        pltpu.VMEM((2 * _EP,), jnp.int32),
        pltpu.VMEM((_ROWS * 2 * _EP,), jnp.float32),
    ],
)(_sc_build_operators)


def _fused_body(ops_ref, x_ref,
                wl0, bl0, wr0, wl1, bl1, wr1, wl2, bl2, wr2,
                wl3, bl3, wr3, wl4, bl4, wr4, out_ref):
    ops = ops_ref[...]
    p_src_t = ops[:, :_EP].astype(jnp.bfloat16)
    p_dst = ops[:, _EP:]
    count = jnp.sum(p_dst, axis=1, keepdims=True)
    p_dst = (p_dst / jnp.maximum(count, 1.0)).astype(jnp.bfloat16)

    weights = ((wl0, bl0, wr0), (wl1, bl1, wr1), (wl2, bl2, wr2),
               (wl3, bl3, wr3), (wl4, bl4, wr4))

    h = jnp.concatenate(
        [x_ref[...], jnp.zeros((_NP - _N, 128), jnp.float32)], axis=0)
    for i, (wl, bl, wr) in enumerate(weights):
        if i > 0:
            h = jnp.maximum(h, 0.0)
        hb = h.astype(jnp.bfloat16)
        xj = lax.dot_general(p_src_t, hb, (((0,), (0,)), ((), ())),
                             preferred_element_type=jnp.float32)
        m = lax.dot_general(xj.astype(jnp.bfloat16),
                            wl[...].astype(jnp.bfloat16),
                            (((1,), (1,)), ((), ())),
                            preferred_element_type=jnp.float32)
        aggl = lax.dot_general(p_dst, m.astype(jnp.bfloat16),
                               (((1,), (0,)), ((), ())),
                               preferred_element_type=jnp.float32)
        dense = lax.dot_general(hb, wr[...].astype(jnp.bfloat16),
                                (((1,), (1,)), ((), ())),
                                preferred_element_type=jnp.float32)
        h = aggl + dense + bl[...]
    out_ref[...] = h[:_N]


def kernel(L_x_, L_edge_index_, L_self_modules_convs_modules_0_modules_lin_l_parameters_weight_, L_self_modules_convs_modules_0_modules_lin_l_parameters_bias_, L_self_modules_convs_modules_0_modules_lin_r_parameters_weight_, L_self_modules_convs_modules_1_modules_lin_l_parameters_weight_, L_self_modules_convs_modules_1_modules_lin_l_parameters_bias_, L_self_modules_convs_modules_1_modules_lin_r_parameters_weight_, L_self_modules_convs_modules_2_modules_lin_l_parameters_weight_, L_self_modules_convs_modules_2_modules_lin_l_parameters_bias_, L_self_modules_convs_modules_2_modules_lin_r_parameters_weight_, L_self_modules_convs_modules_3_modules_lin_l_parameters_weight_, L_self_modules_convs_modules_3_modules_lin_l_parameters_bias_, L_self_modules_convs_modules_3_modules_lin_r_parameters_weight_, L_self_modules_convs_modules_4_modules_lin_l_parameters_weight_, L_self_modules_convs_modules_4_modules_lin_l_parameters_bias_, L_self_modules_convs_modules_4_modules_lin_r_parameters_weight_):
    x = L_x_
    edge_p = jnp.pad(L_edge_index_, ((0, 0), (0, _EP - _E)),
                     constant_values=-1)
    ops_flat = _sc_prep(edge_p.reshape(-1))
    ops = ops_flat.reshape(_NP, 2 * _EP)

    ws = [
        L_self_modules_convs_modules_0_modules_lin_l_parameters_weight_,
        L_self_modules_convs_modules_0_modules_lin_l_parameters_bias_.reshape(1, -1),
        L_self_modules_convs_modules_0_modules_lin_r_parameters_weight_,
        L_self_modules_convs_modules_1_modules_lin_l_parameters_weight_,
        L_self_modules_convs_modules_1_modules_lin_l_parameters_bias_.reshape(1, -1),
        L_self_modules_convs_modules_1_modules_lin_r_parameters_weight_,
        L_self_modules_convs_modules_2_modules_lin_l_parameters_weight_,
        L_self_modules_convs_modules_2_modules_lin_l_parameters_bias_.reshape(1, -1),
        L_self_modules_convs_modules_2_modules_lin_r_parameters_weight_,
        L_self_modules_convs_modules_3_modules_lin_l_parameters_weight_,
        L_self_modules_convs_modules_3_modules_lin_l_parameters_bias_.reshape(1, -1),
        L_self_modules_convs_modules_3_modules_lin_r_parameters_weight_,
        L_self_modules_convs_modules_4_modules_lin_l_parameters_weight_,
        L_self_modules_convs_modules_4_modules_lin_l_parameters_bias_.reshape(1, -1),
        L_self_modules_convs_modules_4_modules_lin_r_parameters_weight_,
    ]

    return pl.pallas_call(
        _fused_body,
        out_shape=jax.ShapeDtypeStruct((_N, 256), jnp.float32),
    )(ops, x, *ws)
